# baseline (speedup 1.0000x reference)
import functools

import jax
import jax.numpy as jnp
from jax import lax
from jax.experimental import pallas as pl
from jax.experimental.pallas import tpu as pltpu
import jax.experimental.pallas.tpu_sc as plsc
import numpy as np

N = 10000
L = 16
E = 320000
NC, NS = 2, 16
NW = NC * NS
CB = 128
NCHW = 80
EP = NW * NCHW * CB
ZB = 624
ZTAIL = N - NS * ZB
NB = 5


@functools.cache
def _make_spmv(edge_sets, combine):
  mesh = plsc.VectorSubcoreMesh(
      core_axis_name="c", subcore_axis_name="s", num_cores=NC, num_subcores=NS)
  ncw = edge_sets * NCHW
  scratch = [
      pltpu.VMEM((ncw, CB), jnp.int32),
      pltpu.VMEM((ncw, CB), jnp.int32),
      pltpu.VMEM((ncw, CB), jnp.float32),
      pltpu.VMEM((NB, CB, L), jnp.float32),
      pltpu.VMEM((NB, CB, L), jnp.float32),
      pltpu.VMEM((ZB, L), jnp.float32),
      pltpu.VMEM((ZB, L), jnp.float32),
      pltpu.VMEM_SHARED((edge_sets * N, L), jnp.float32),
  ]
  scratch += [pltpu.SemaphoreType.DMA for _ in range(2 * NB)]

  out_type = [jax.ShapeDtypeStruct((NC, N, L), jnp.float32)] * edge_sets
  if combine:
    out_type = out_type + [jax.ShapeDtypeStruct((N, L), jnp.float32)]
  single_out = len(out_type) == 1
  if single_out:
    out_type = out_type[0]

  @functools.partial(
      pl.kernel,
      out_type=out_type if single_out else tuple(out_type),
      mesh=mesh,
      scratch_types=scratch,
      compiler_params=pltpu.CompilerParams(use_tc_tiling_on_sc=False),
  )
  def spmv(*refs):
    it = iter(refs)
    prev = next(it)
    cols_h, rows_h, vals_h = next(it), next(it), next(it)
    outs = [next(it) for _ in range(edge_sets)]
    comb = next(it) if combine else prev
    colv, rowv, valv = next(it), next(it), next(it)
    gbuf, scaled, zbuf, cbuf = next(it), next(it), next(it), next(it)
    acc = next(it)
    gsem = [next(it) for _ in range(NB)]
    ssem = [next(it) for _ in range(NB)]

    cid = lax.axis_index("c")
    sid = lax.axis_index("s")
    wid = sid * NC + cid
    base = sid * ZB
    tailb = NS * ZB

    def zero_row(i, _):
      zbuf[i, :] = jnp.zeros((L,), jnp.float32)
      return 0
    lax.fori_loop(0, ZB, zero_row, 0)
    for es in range(edge_sets):
      pltpu.sync_copy(zbuf, acc.at[pl.ds(es * N + base, ZB), :])
      @pl.when(sid == NS - 1)
      def _():
        pltpu.sync_copy(zbuf.at[pl.ds(0, ZTAIL), :],
                        acc.at[pl.ds(es * N + tailb, ZTAIL), :])

    if combine:
      def merge(rows, roff, abuf, bbuf):
        pltpu.sync_copy(prev.at[0, pl.ds(roff, rows), :], abuf)
        pltpu.sync_copy(prev.at[1, pl.ds(roff, rows), :], bbuf)
        def add_row(i, _):
          abuf[i, :] = abuf[i, :] + bbuf[i, :]
          return 0
        lax.fori_loop(0, rows, add_row, 0)
        pltpu.sync_copy(abuf, comb.at[pl.ds(roff, rows), :])
      merge(ZB, base, zbuf, cbuf)
      @pl.when(sid == NS - 1)
      def _():
        merge(ZTAIL, tailb, zbuf.at[pl.ds(0, ZTAIL), :],
              cbuf.at[pl.ds(0, ZTAIL), :])
    plsc.subcore_barrier()

    pltpu.sync_copy(cols_h.at[wid], colv)
    pltpu.sync_copy(rows_h.at[wid], rowv)
    pltpu.sync_copy(vals_h.at[wid], valv)

    def fire_gather(j, b):
      pltpu.async_copy(comb.at[colv.at[j]], gbuf.at[b], gsem[b])

    for b in range(NB):
      fire_gather(b, b)

    T = ncw // NB

    def outer(t, _):
      for b in range(NB):
        j = t * NB + b
        pltpu.make_async_copy(comb.at[colv.at[j]], gbuf.at[b], gsem[b]).wait()

        @pl.when(t > 0)
        def _():
          pltpu.make_async_copy(
              scaled.at[b], acc.at[rowv.at[0]], ssem[b]).wait()

        for grp in range(CB // L):
          vv = valv[j, pl.ds(grp * L, L)]
          for e in range(L):
            idx = grp * L + e
            scaled[b, idx, :] = gbuf[b, idx, :] * vv[e]
        pltpu.async_copy(scaled.at[b], acc.at[rowv.at[j]], ssem[b], add=True)

        @pl.when(t < T - 1)
        def _():
          fire_gather(j + NB, b)
      return 0
    lax.fori_loop(0, T, outer, 0)

    for b in range(NB):
      pltpu.make_async_copy(scaled.at[b], acc.at[rowv.at[0]], ssem[b]).wait()

    plsc.subcore_barrier()
    for es in range(edge_sets):
      pltpu.sync_copy(acc.at[pl.ds(es * N + base, ZB), :],
                      outs[es].at[cid, pl.ds(base, ZB), :])
      @pl.when(sid == NS - 1)
      def _():
        pltpu.sync_copy(acc.at[pl.ds(es * N + tailb, ZTAIL), :],
                        outs[es].at[cid, pl.ds(tailb, ZTAIL), :])

  return spmv


TC_G = 10
TC_NB = N // TC_G


def _tc_body(x_ref, c1_ref, s2_ref, c3_ref, c4_ref, s5_ref,
             w_ref, bu_ref, bc_ref, hx_ref, out_ref, acc_ref):
  i = pl.program_id(0)

  @pl.when(i == 0)
  def _():
    acc_ref[...] = jnp.zeros_like(acc_ref)

  s2 = s2_ref[0] + s2_ref[1]
  s5 = s5_ref[0] + s5_ref[1]
  x = x_ref[...]
  c1 = c1_ref[...]
  c3 = c3_ref[...]
  c4 = c4_ref[...]
  X = jnp.concatenate(
      [x, c1, 2.0 * s2 - x, c3, 2.0 * c4 - c1, c4, 2.0 * s5 - c3], axis=1)
  dn = (((0,), (0,)), ((), ()))
  acc_ref[...] += lax.dot_general(
      X, w_ref[...], dn, preferred_element_type=jnp.float32)

  @pl.when(i == TC_G - 1)
  def _():
    acc = acc_ref[...]
    u = jax.nn.sigmoid(acc[:, :64] + bu_ref[...])
    c = jnp.tanh(acc[:, 64:] + bc_ref[...])
    out_ref[...] = u * hx_ref[...] + (1.0 - u) * c


_IDX = np.array([(m % 7) * 16 + m // 7 for m in range(112)], dtype=np.int32)
_INV = np.array([(q % 16) * 7 + q // 16 for q in range(112)], dtype=np.int32)


def kernel(inputs, hx, sup0_rows, sup0_cols, sup0_vals,
           sup1_rows, sup1_cols, sup1_vals, W_ru, b_ru, W_c, b_c):
  x = jnp.transpose(inputs, (1, 2, 0)).reshape(N, L)

  def prep(a, dtype):
    a = a.astype(dtype)
    return jnp.concatenate(
        [a, jnp.zeros((EP - E,), dtype)]).reshape(NW, NCHW, CB)

  c0 = prep(sup0_cols, jnp.int32)
  r0 = prep(sup0_rows, jnp.int32)
  v0 = prep(sup0_vals, jnp.float32)
  c1e = prep(sup1_cols, jnp.int32)
  r1e = prep(sup1_rows, jnp.int32)
  v1e = prep(sup1_vals, jnp.float32)

  cc = jnp.concatenate([c0, c1e], axis=1)
  rc = jnp.concatenate([r0, r1e + N], axis=1)
  vc = jnp.concatenate([v0, v1e], axis=1)

  spmv_x = _make_spmv(1, False)
  spmv_d = _make_spmv(2, True)
  spmv_s = _make_spmv(1, True)
  S1 = spmv_x(x, c0, r0, v0)
  S2, S3, comb1 = spmv_d(S1, cc, rc, vc)
  S4, comb3 = spmv_s(S3, c1e, r1e, v1e)
  S5, comb4 = spmv_s(S4, c1e, r1e, v1e)

  w = jnp.concatenate([W_ru[:, 64:], W_c], axis=1)
  bu = b_ru[64:].reshape(1, 64)
  bc = b_c.reshape(1, 64)
  hxp = hx[_INV]

  nblk = pl.BlockSpec((TC_NB, L), lambda i: (i, 0))
  pblk = pl.BlockSpec((NC, TC_NB, L), lambda i: (0, i, 0))
  new_big = pl.pallas_call(
      _tc_body,
      grid=(TC_G,),
      in_specs=[
          nblk, nblk, pblk, nblk, nblk, pblk,
          pl.BlockSpec((TC_NB, 128), lambda i: (i, 0)),
          pl.BlockSpec((1, 64), lambda i: (0, 0)),
          pl.BlockSpec((1, 64), lambda i: (0, 0)),
          pl.BlockSpec((112, 64), lambda i: (0, 0)),
      ],
      out_specs=pl.BlockSpec((112, 64), lambda i: (0, 0)),
      out_shape=jax.ShapeDtypeStruct((112, 64), jnp.float32),
      scratch_shapes=[pltpu.VMEM((112, 128), jnp.float32)],
  )(x, comb1, S2, comb3, comb4, S5, w, bu, bc, hxp)
  return new_big[_IDX]


# speedup vs baseline: 1.9203x; 1.9203x over previous
import functools

import jax
import jax.numpy as jnp
from jax import lax
from jax.experimental import pallas as pl
from jax.experimental.pallas import tpu as pltpu
import jax.experimental.pallas.tpu_sc as plsc
import numpy as np

N = 10000
L = 16
E = 320000
NC, NS = 2, 16
NW = NC * NS
CB = 128
NCHW = 80
EP = NW * NCHW * CB
ZB = 624
ZTAIL = N - NS * ZB
NB = 5


@functools.cache
def _make_spmv(edge_sets, combine):
  mesh = plsc.VectorSubcoreMesh(
      core_axis_name="c", subcore_axis_name="s", num_cores=NC, num_subcores=NS)
  ncw = edge_sets * NCHW
  scratch = [
      pltpu.VMEM((ncw, CB), jnp.int32),
      pltpu.VMEM((ncw, CB), jnp.int32),
      pltpu.VMEM((ncw, CB), jnp.float32),
      pltpu.VMEM((NB, CB, L), jnp.float32),
      pltpu.VMEM((NB, CB, L), jnp.float32),
      pltpu.VMEM((ZB, L), jnp.float32),
      pltpu.VMEM((ZB, L), jnp.float32),
      pltpu.VMEM_SHARED((edge_sets * N, L), jnp.float32),
  ]
  scratch += [pltpu.SemaphoreType.DMA for _ in range(2 * NB)]

  out_type = [jax.ShapeDtypeStruct((NC, N, L), jnp.float32)] * edge_sets
  if combine:
    out_type = out_type + [jax.ShapeDtypeStruct((N, L), jnp.float32)]
  single_out = len(out_type) == 1
  if single_out:
    out_type = out_type[0]

  @functools.partial(
      pl.kernel,
      out_type=out_type if single_out else tuple(out_type),
      mesh=mesh,
      scratch_types=scratch,
      compiler_params=pltpu.CompilerParams(use_tc_tiling_on_sc=False),
  )
  def spmv(*refs):
    it = iter(refs)
    prev = next(it)
    cols_h, rows_h, vals_h = next(it), next(it), next(it)
    outs = [next(it) for _ in range(edge_sets)]
    comb = next(it) if combine else prev
    colv, rowv, valv = next(it), next(it), next(it)
    gbuf, scaled, zbuf, cbuf = next(it), next(it), next(it), next(it)
    acc = next(it)
    gsem = [next(it) for _ in range(NB)]
    ssem = [next(it) for _ in range(NB)]

    cid = lax.axis_index("c")
    sid = lax.axis_index("s")
    wid = sid * NC + cid
    base = sid * ZB
    tailb = NS * ZB

    def zero_row(i, _):
      zbuf[i, :] = jnp.zeros((L,), jnp.float32)
      return 0
    lax.fori_loop(0, ZB, zero_row, 0)
    for es in range(edge_sets):
      pltpu.sync_copy(zbuf, acc.at[pl.ds(es * N + base, ZB), :])
      @pl.when(sid == NS - 1)
      def _():
        pltpu.sync_copy(zbuf.at[pl.ds(0, ZTAIL), :],
                        acc.at[pl.ds(es * N + tailb, ZTAIL), :])

    if combine:
      def merge(rows, roff, abuf, bbuf):
        pltpu.sync_copy(prev.at[0, pl.ds(roff, rows), :], abuf)
        pltpu.sync_copy(prev.at[1, pl.ds(roff, rows), :], bbuf)
        def add_row(i, _):
          abuf[i, :] = abuf[i, :] + bbuf[i, :]
          return 0
        lax.fori_loop(0, rows, add_row, 0)
        pltpu.sync_copy(abuf, comb.at[pl.ds(roff, rows), :])
      merge(ZB, base, zbuf, cbuf)
      @pl.when(sid == NS - 1)
      def _():
        merge(ZTAIL, tailb, zbuf.at[pl.ds(0, ZTAIL), :],
              cbuf.at[pl.ds(0, ZTAIL), :])
    plsc.subcore_barrier()

    pltpu.sync_copy(cols_h.at[wid], colv)
    pltpu.sync_copy(rows_h.at[wid], rowv)
    pltpu.sync_copy(vals_h.at[wid], valv)

    def fire_gather(j, b):
      pltpu.async_copy(comb.at[colv.at[j]], gbuf.at[b], gsem[b])

    for b in range(NB):
      fire_gather(b, b)

    T = ncw // NB

    def outer(t, _):
      for b in range(NB):
        j = t * NB + b
        pltpu.make_async_copy(comb.at[colv.at[j]], gbuf.at[b], gsem[b]).wait()

        @pl.when(t > 0)
        def _():
          pltpu.make_async_copy(
              scaled.at[b], acc.at[rowv.at[0]], ssem[b]).wait()

        for grp in range(CB // L):
          vv = valv[j, pl.ds(grp * L, L)]
          for e in range(L):
            idx = grp * L + e
            scaled[b, idx, :] = gbuf[b, idx, :] * vv[e]
        pltpu.async_copy(scaled.at[b], acc.at[rowv.at[j]], ssem[b], add=True)

        @pl.when(t < T - 1)
        def _():
          fire_gather(j + NB, b)
      return 0
    lax.fori_loop(0, T, outer, 0)

    for b in range(NB):
      pltpu.make_async_copy(scaled.at[b], acc.at[rowv.at[0]], ssem[b]).wait()

    plsc.subcore_barrier()
    for es in range(edge_sets):
      pltpu.sync_copy(acc.at[pl.ds(es * N + base, ZB), :],
                      outs[es].at[cid, pl.ds(base, ZB), :])
      @pl.when(sid == NS - 1)
      def _():
        pltpu.sync_copy(acc.at[pl.ds(es * N + tailb, ZTAIL), :],
                        outs[es].at[cid, pl.ds(tailb, ZTAIL), :])

  return spmv


TC_G = 10
TC_NB = N // TC_G


def _tc_body(x_ref, c1_ref, s2_ref, c3_ref, c4_ref, s5_ref,
             w_ref, bu_ref, bc_ref, hx_ref, out_ref, acc_ref):
  i = pl.program_id(0)

  @pl.when(i == 0)
  def _():
    acc_ref[...] = jnp.zeros_like(acc_ref)

  s2 = s2_ref[0] + s2_ref[1]
  s5 = s5_ref[0] + s5_ref[1]
  x = x_ref[...]
  c1 = c1_ref[...]
  c3 = c3_ref[...]
  c4 = c4_ref[...]
  X = jnp.concatenate(
      [x, c1, 2.0 * s2 - x, c3, 2.0 * c4 - c1, c4, 2.0 * s5 - c3], axis=1)
  dn = (((0,), (0,)), ((), ()))
  acc_ref[...] += lax.dot_general(
      X, w_ref[...], dn, preferred_element_type=jnp.float32)

  @pl.when(i == TC_G - 1)
  def _():
    acc = acc_ref[...]
    u = jax.nn.sigmoid(acc[:, :64] + bu_ref[...])
    c = jnp.tanh(acc[:, 64:] + bc_ref[...])
    out_ref[...] = u * hx_ref[...] + (1.0 - u) * c


_IDX = np.array([(m % 7) * 16 + m // 7 for m in range(112)], dtype=np.int32)
_INV = np.array([(q % 16) * 7 + q // 16 for q in range(112)], dtype=np.int32)


def kernel(inputs, hx, sup0_rows, sup0_cols, sup0_vals,
           sup1_rows, sup1_cols, sup1_vals, W_ru, b_ru, W_c, b_c):
  x = jnp.transpose(inputs, (1, 2, 0)).reshape(N, L)

  pad_idx = (jnp.arange(EP - E, dtype=jnp.int32) * 13) % N

  def prep(a, dtype):
    a = a.astype(dtype)
    if dtype == jnp.int32:
      pad = pad_idx
    else:
      pad = jnp.zeros((EP - E,), dtype)
    return jnp.concatenate([a, pad]).reshape(NW, NCHW, CB)

  c0 = prep(sup0_cols, jnp.int32)
  r0 = prep(sup0_rows, jnp.int32)
  v0 = prep(sup0_vals, jnp.float32)
  c1e = prep(sup1_cols, jnp.int32)
  r1e = prep(sup1_rows, jnp.int32)
  v1e = prep(sup1_vals, jnp.float32)

  cc = jnp.concatenate([c0, c1e], axis=1)
  rc = jnp.concatenate([r0, r1e + N], axis=1)
  vc = jnp.concatenate([v0, v1e], axis=1)

  spmv_x = _make_spmv(1, False)
  spmv_d = _make_spmv(2, True)
  spmv_s = _make_spmv(1, True)
  S1 = spmv_x(x, c0, r0, v0)
  S2, S3, comb1 = spmv_d(S1, cc, rc, vc)
  S4, comb3 = spmv_s(S3, c1e, r1e, v1e)
  S5, comb4 = spmv_s(S4, c1e, r1e, v1e)

  w = jnp.concatenate([W_ru[:, 64:], W_c], axis=1)
  bu = b_ru[64:].reshape(1, 64)
  bc = b_c.reshape(1, 64)
  hxp = hx[_INV]

  nblk = pl.BlockSpec((TC_NB, L), lambda i: (i, 0))
  pblk = pl.BlockSpec((NC, TC_NB, L), lambda i: (0, i, 0))
  new_big = pl.pallas_call(
      _tc_body,
      grid=(TC_G,),
      in_specs=[
          nblk, nblk, pblk, nblk, nblk, pblk,
          pl.BlockSpec((TC_NB, 128), lambda i: (i, 0)),
          pl.BlockSpec((1, 64), lambda i: (0, 0)),
          pl.BlockSpec((1, 64), lambda i: (0, 0)),
          pl.BlockSpec((112, 64), lambda i: (0, 0)),
      ],
      out_specs=pl.BlockSpec((112, 64), lambda i: (0, 0)),
      out_shape=jax.ShapeDtypeStruct((112, 64), jnp.float32),
      scratch_shapes=[pltpu.VMEM((112, 128), jnp.float32)],
  )(x, comb1, S2, comb3, comb4, S5, w, bu, bc, hxp)
  return new_big[_IDX]


# speedup vs baseline: 1.9376x; 1.0090x over previous
import functools

import jax
import jax.numpy as jnp
from jax import lax
from jax.experimental import pallas as pl
from jax.experimental.pallas import tpu as pltpu
import jax.experimental.pallas.tpu_sc as plsc
import numpy as np

N = 10000
L = 16
E = 320000
NC, NS = 2, 16
NW = NC * NS
CB = 128
NCHW = 80
EP = NW * NCHW * CB
ZB = 624
ZTAIL = N - NS * ZB
NB = 5


@functools.cache
def _make_spmv(edge_sets, combine):
  mesh = plsc.VectorSubcoreMesh(
      core_axis_name="c", subcore_axis_name="s", num_cores=NC, num_subcores=NS)
  ncw = edge_sets * NCHW
  scratch = [
      pltpu.VMEM((ncw, CB), jnp.int32),
      pltpu.VMEM((ncw, CB), jnp.int32),
      pltpu.VMEM((ncw, CB), jnp.float32),
      pltpu.VMEM((NB, CB, L), jnp.float32),
      pltpu.VMEM((NB, CB, L), jnp.float32),
      pltpu.VMEM((ZB, L), jnp.float32),
      pltpu.VMEM((ZB, L), jnp.float32),
      pltpu.VMEM_SHARED((edge_sets * N, L), jnp.float32),
  ]
  scratch += [pltpu.SemaphoreType.DMA for _ in range(2 * NB)]

  out_type = [jax.ShapeDtypeStruct((NC, N, L), jnp.float32)] * edge_sets
  if combine:
    out_type = out_type + [jax.ShapeDtypeStruct((N, L), jnp.float32)]
  single_out = len(out_type) == 1
  if single_out:
    out_type = out_type[0]

  @functools.partial(
      pl.kernel,
      out_type=out_type if single_out else tuple(out_type),
      mesh=mesh,
      scratch_types=scratch,
      compiler_params=pltpu.CompilerParams(use_tc_tiling_on_sc=False),
  )
  def spmv(*refs):
    it = iter(refs)
    prev = next(it)
    esets = [(next(it), next(it), next(it)) for _ in range(edge_sets)]
    outs = [next(it) for _ in range(edge_sets)]
    comb = next(it) if combine else prev
    colv, rowv, valv = next(it), next(it), next(it)
    gbuf, scaled, zbuf, cbuf = next(it), next(it), next(it), next(it)
    acc = next(it)
    gsem = [next(it) for _ in range(NB)]
    ssem = [next(it) for _ in range(NB)]

    cid = lax.axis_index("c")
    sid = lax.axis_index("s")
    wid = sid * NC + cid
    base = sid * ZB
    tailb = NS * ZB

    def zero_row(i, _):
      zbuf[i, :] = jnp.zeros((L,), jnp.float32)
      return 0
    lax.fori_loop(0, ZB, zero_row, 0)
    for es in range(edge_sets):
      pltpu.sync_copy(zbuf, acc.at[pl.ds(es * N + base, ZB), :])
      @pl.when(sid == NS - 1)
      def _():
        pltpu.sync_copy(zbuf.at[pl.ds(0, ZTAIL), :],
                        acc.at[pl.ds(es * N + tailb, ZTAIL), :])

    if combine:
      def merge(rows, roff, abuf, bbuf):
        pltpu.sync_copy(prev.at[0, pl.ds(roff, rows), :], abuf)
        pltpu.sync_copy(prev.at[1, pl.ds(roff, rows), :], bbuf)
        def add_row(i, _):
          abuf[i, :] = abuf[i, :] + bbuf[i, :]
          return 0
        lax.fori_loop(0, rows, add_row, 0)
        pltpu.sync_copy(abuf, comb.at[pl.ds(roff, rows), :])
      merge(ZB, base, zbuf, cbuf)
      @pl.when(sid == NS - 1)
      def _():
        merge(ZTAIL, tailb, zbuf.at[pl.ds(0, ZTAIL), :],
              cbuf.at[pl.ds(0, ZTAIL), :])
    plsc.subcore_barrier()

    for es, (ch, rh, vh) in enumerate(esets):
      pltpu.sync_copy(ch.at[wid], colv.at[pl.ds(es * NCHW, NCHW), :])
      pltpu.sync_copy(rh.at[wid], rowv.at[pl.ds(es * NCHW, NCHW), :])
      pltpu.sync_copy(vh.at[wid], valv.at[pl.ds(es * NCHW, NCHW), :])

    def fire_gather(j, b):
      pltpu.async_copy(comb.at[colv.at[j]], gbuf.at[b], gsem[b])

    for b in range(NB):
      fire_gather(b, b)

    T = ncw // NB

    def outer(t, _):
      for b in range(NB):
        j = t * NB + b
        pltpu.make_async_copy(comb.at[colv.at[j]], gbuf.at[b], gsem[b]).wait()

        @pl.when(t > 0)
        def _():
          pltpu.make_async_copy(
              scaled.at[b], acc.at[rowv.at[0]], ssem[b]).wait()

        for grp in range(CB // L):
          vv = valv[j, pl.ds(grp * L, L)]
          for e in range(L):
            idx = grp * L + e
            scaled[b, idx, :] = gbuf[b, idx, :] * vv[e]
        pltpu.async_copy(scaled.at[b], acc.at[rowv.at[j]], ssem[b], add=True)

        @pl.when(t < T - 1)
        def _():
          fire_gather(j + NB, b)
      return 0
    lax.fori_loop(0, T, outer, 0)

    for b in range(NB):
      pltpu.make_async_copy(scaled.at[b], acc.at[rowv.at[0]], ssem[b]).wait()

    plsc.subcore_barrier()
    for es in range(edge_sets):
      pltpu.sync_copy(acc.at[pl.ds(es * N + base, ZB), :],
                      outs[es].at[cid, pl.ds(base, ZB), :])
      @pl.when(sid == NS - 1)
      def _():
        pltpu.sync_copy(acc.at[pl.ds(es * N + tailb, ZTAIL), :],
                        outs[es].at[cid, pl.ds(tailb, ZTAIL), :])

  return spmv


TC_G = 10
TC_NB = N // TC_G


def _tc_body(x_ref, c1_ref, s2_ref, c3_ref, c4_ref, s5_ref,
             wu_ref, wc_ref, bu_ref, bc_ref, hx_ref, out_ref, acc_ref):
  i = pl.program_id(0)

  @pl.when(i == 0)
  def _():
    acc_ref[...] = jnp.zeros_like(acc_ref)

  s2 = s2_ref[0] + s2_ref[1]
  s5 = s5_ref[0] + s5_ref[1]
  x = x_ref[...]
  c1 = c1_ref[...]
  c3 = c3_ref[...]
  c4 = c4_ref[...]
  X = jnp.concatenate(
      [x, c1, 2.0 * s2 - x, c3, 2.0 * c4 - c1, c4, 2.0 * s5 - c3], axis=1)
  dn = (((0,), (0,)), ((), ()))
  acc_ref[:, 0:64] += lax.dot_general(
      X, wu_ref[:, 64:128], dn, preferred_element_type=jnp.float32)
  acc_ref[:, 64:128] += lax.dot_general(
      X, wc_ref[...], dn, preferred_element_type=jnp.float32)

  @pl.when(i == TC_G - 1)
  def _():
    acc = acc_ref[...]
    u = jax.nn.sigmoid(acc[:, :64] + bu_ref[...])
    c = jnp.tanh(acc[:, 64:] + bc_ref[...])
    out_ref[...] = u * hx_ref[...] + (1.0 - u) * c


_IDX = np.array([(m % 7) * 16 + m // 7 for m in range(112)], dtype=np.int32)
_INV = np.array([(q % 16) * 7 + q // 16 for q in range(112)], dtype=np.int32)


def kernel(inputs, hx, sup0_rows, sup0_cols, sup0_vals,
           sup1_rows, sup1_cols, sup1_vals, W_ru, b_ru, W_c, b_c):
  x = jnp.transpose(inputs, (1, 2, 0)).reshape(N, L)

  pad_idx = (jnp.arange(EP - E, dtype=jnp.int32) * 13) % N

  def prep(a, dtype):
    a = a.astype(dtype)
    if dtype == jnp.int32:
      pad = pad_idx
    else:
      pad = jnp.zeros((EP - E,), dtype)
    return jnp.concatenate([a, pad]).reshape(NW, NCHW, CB)

  c0 = prep(sup0_cols, jnp.int32)
  r0 = prep(sup0_rows, jnp.int32)
  v0 = prep(sup0_vals, jnp.float32)
  c1e = prep(sup1_cols, jnp.int32)
  r1e = prep(sup1_rows, jnp.int32)
  v1e = prep(sup1_vals, jnp.float32)

  spmv_x = _make_spmv(1, False)
  spmv_d = _make_spmv(2, True)
  spmv_s = _make_spmv(1, True)
  S1 = spmv_x(x, c0, r0, v0)
  S2, S3, comb1 = spmv_d(S1, c0, r0, v0, c1e, r1e + N, v1e)
  S4, comb3 = spmv_s(S3, c1e, r1e, v1e)
  S5, comb4 = spmv_s(S4, c1e, r1e, v1e)

  bu = b_ru[64:].reshape(1, 64)
  bc = b_c.reshape(1, 64)
  hxp = hx[_INV]

  nblk = pl.BlockSpec((TC_NB, L), lambda i: (i, 0))
  pblk = pl.BlockSpec((NC, TC_NB, L), lambda i: (0, i, 0))
  new_big = pl.pallas_call(
      _tc_body,
      grid=(TC_G,),
      in_specs=[
          nblk, nblk, pblk, nblk, nblk, pblk,
          pl.BlockSpec((TC_NB, 128), lambda i: (i, 0)),
          pl.BlockSpec((TC_NB, 64), lambda i: (i, 0)),
          pl.BlockSpec((1, 64), lambda i: (0, 0)),
          pl.BlockSpec((1, 64), lambda i: (0, 0)),
          pl.BlockSpec((112, 64), lambda i: (0, 0)),
      ],
      out_specs=pl.BlockSpec((112, 64), lambda i: (0, 0)),
      out_shape=jax.ShapeDtypeStruct((112, 64), jnp.float32),
      scratch_shapes=[pltpu.VMEM((112, 128), jnp.float32)],
  )(x, comb1, S2, comb3, comb4, S5, W_ru, W_c, bu, bc, hxp)
  return new_big[_IDX]


# speedup vs baseline: 1.9639x; 1.0136x over previous
import functools

import jax
import jax.numpy as jnp
from jax import lax
from jax.experimental import pallas as pl
from jax.experimental.pallas import tpu as pltpu
import jax.experimental.pallas.tpu_sc as plsc
import numpy as np

N = 10000
L = 16
E = 320000
NC, NS = 2, 16
NW = NC * NS
CB = 128
NCHW = 80
EP = NW * NCHW * CB
ZB = 624
ZTAIL = N - NS * ZB
NB = 5


@functools.cache
def _make_spmv(edge_sets, combine):
  mesh = plsc.VectorSubcoreMesh(
      core_axis_name="c", subcore_axis_name="s", num_cores=NC, num_subcores=NS)
  ncw = edge_sets * NCHW
  scratch = [
      pltpu.VMEM((ncw, CB), jnp.int32),
      pltpu.VMEM((ncw, CB), jnp.int32),
      pltpu.VMEM((ncw, CB), jnp.float32),
      pltpu.VMEM((NB, CB, L), jnp.float32),
      pltpu.VMEM((NB, CB, L), jnp.float32),
      pltpu.VMEM((ZB, L), jnp.float32),
      pltpu.VMEM((ZB, L), jnp.float32),
      pltpu.VMEM_SHARED((edge_sets * N, L), jnp.float32),
  ]
  scratch += [pltpu.SemaphoreType.DMA for _ in range(2 * NB)]

  out_type = [jax.ShapeDtypeStruct((NC, N, L), jnp.float32)] * edge_sets
  if combine:
    out_type = out_type + [jax.ShapeDtypeStruct((N, L), jnp.float32)]
  single_out = len(out_type) == 1
  if single_out:
    out_type = out_type[0]

  @functools.partial(
      pl.kernel,
      out_type=out_type if single_out else tuple(out_type),
      mesh=mesh,
      scratch_types=scratch,
      compiler_params=pltpu.CompilerParams(use_tc_tiling_on_sc=False),
  )
  def spmv(*refs):
    it = iter(refs)
    prev = next(it)
    esets = [(next(it), next(it), next(it)) for _ in range(edge_sets)]
    outs = [next(it) for _ in range(edge_sets)]
    comb = next(it) if combine else prev
    colv, rowv, valv = next(it), next(it), next(it)
    gbuf, scaled, zbuf, cbuf = next(it), next(it), next(it), next(it)
    acc = next(it)
    gsem = [next(it) for _ in range(NB)]
    ssem = [next(it) for _ in range(NB)]

    cid = lax.axis_index("c")
    sid = lax.axis_index("s")
    wid = sid * NC + cid
    base = sid * ZB
    tailb = NS * ZB

    def zero_row(i, _):
      zbuf[i, :] = jnp.zeros((L,), jnp.float32)
      return 0
    lax.fori_loop(0, ZB, zero_row, 0)
    for es in range(edge_sets):
      pltpu.sync_copy(zbuf, acc.at[pl.ds(es * N + base, ZB), :])
      @pl.when(sid == NS - 1)
      def _():
        pltpu.sync_copy(zbuf.at[pl.ds(0, ZTAIL), :],
                        acc.at[pl.ds(es * N + tailb, ZTAIL), :])

    if combine:
      def merge(rows, roff, abuf, bbuf):
        pltpu.sync_copy(prev.at[0, pl.ds(roff, rows), :], abuf)
        pltpu.sync_copy(prev.at[1, pl.ds(roff, rows), :], bbuf)
        def add_row(i, _):
          abuf[i, :] = abuf[i, :] + bbuf[i, :]
          return 0
        lax.fori_loop(0, rows, add_row, 0)
        pltpu.sync_copy(abuf, comb.at[pl.ds(roff, rows), :])
      merge(ZB, base, zbuf, cbuf)
      @pl.when(sid == NS - 1)
      def _():
        merge(ZTAIL, tailb, zbuf.at[pl.ds(0, ZTAIL), :],
              cbuf.at[pl.ds(0, ZTAIL), :])
    plsc.subcore_barrier()

    stage = []
    for es, (ch, rh, vh) in enumerate(esets):
      sl = pl.ds(es * NCHW, NCHW)
      stage.append(pltpu.async_copy(ch.at[wid], colv.at[sl, :], gsem[0]))
      stage.append(pltpu.async_copy(rh.at[wid], rowv.at[sl, :], gsem[0]))
      stage.append(pltpu.async_copy(vh.at[wid], valv.at[sl, :], gsem[0]))
    for d in stage:
      d.wait()

    def fire_gather(j, b):
      pltpu.async_copy(comb.at[colv.at[j]], gbuf.at[b], gsem[b])

    for b in range(NB):
      fire_gather(b, b)

    T = ncw // NB

    def outer(t, _):
      for b in range(NB):
        j = t * NB + b
        pltpu.make_async_copy(comb.at[colv.at[j]], gbuf.at[b], gsem[b]).wait()

        @pl.when(t > 0)
        def _():
          pltpu.make_async_copy(
              scaled.at[b], acc.at[rowv.at[0]], ssem[b]).wait()

        for grp in range(CB // L):
          vv = valv[j, pl.ds(grp * L, L)]
          for e in range(L):
            idx = grp * L + e
            scaled[b, idx, :] = gbuf[b, idx, :] * vv[e]
        pltpu.async_copy(scaled.at[b], acc.at[rowv.at[j]], ssem[b], add=True)

        @pl.when(t < T - 1)
        def _():
          fire_gather(j + NB, b)
      return 0
    lax.fori_loop(0, T, outer, 0)

    for b in range(NB):
      pltpu.make_async_copy(scaled.at[b], acc.at[rowv.at[0]], ssem[b]).wait()

    plsc.subcore_barrier()
    for es in range(edge_sets):
      pltpu.sync_copy(acc.at[pl.ds(es * N + base, ZB), :],
                      outs[es].at[cid, pl.ds(base, ZB), :])
      @pl.when(sid == NS - 1)
      def _():
        pltpu.sync_copy(acc.at[pl.ds(es * N + tailb, ZTAIL), :],
                        outs[es].at[cid, pl.ds(tailb, ZTAIL), :])

  return spmv


TC_G = 10
TC_NB = N // TC_G


_DN = (((0,), (0,)), ((), ()))


def _accum(acc_ref, X, wu_ref, wc_ref):
  acc_ref[:, 0:64] += lax.dot_general(
      X, wu_ref[:, 64:128], _DN, preferred_element_type=jnp.float32)
  acc_ref[:, 64:128] += lax.dot_general(
      X, wc_ref[...], _DN, preferred_element_type=jnp.float32)


def _tc1_body(x_ref, c1_ref, s2_ref, wu_ref, wc_ref, out_ref, acc_ref):
  i = pl.program_id(0)

  @pl.when(i == 0)
  def _():
    acc_ref[...] = jnp.zeros_like(acc_ref)

  s2 = s2_ref[0] + s2_ref[1]
  x = x_ref[...]
  X = jnp.concatenate([x, c1_ref[...], 2.0 * s2 - x], axis=1)
  _accum(acc_ref, X, wu_ref, wc_ref)

  @pl.when(i == TC_G - 1)
  def _():
    out_ref[...] = acc_ref[...]


def _tc2_body(c1_ref, c3_ref, c4_ref, s5_ref, wu_ref, wc_ref,
              acc1_ref, bu_ref, bc_ref, hx_ref, out_ref, acc_ref):
  i = pl.program_id(0)

  @pl.when(i == 0)
  def _():
    acc_ref[...] = jnp.zeros_like(acc_ref)

  s5 = s5_ref[0] + s5_ref[1]
  c1 = c1_ref[...]
  c3 = c3_ref[...]
  c4 = c4_ref[...]
  X = jnp.concatenate(
      [c3, 2.0 * c4 - c1, c4, 2.0 * s5 - c3], axis=1)
  _accum(acc_ref, X, wu_ref, wc_ref)

  @pl.when(i == TC_G - 1)
  def _():
    acc = jnp.concatenate([acc1_ref[...], acc_ref[...]], axis=0)
    u = jax.nn.sigmoid(acc[:, :64] + bu_ref[...])
    c = jnp.tanh(acc[:, 64:] + bc_ref[...])
    out_ref[...] = u * hx_ref[...] + (1.0 - u) * c


_IDX = np.array([(m % 7) * 16 + m // 7 for m in range(112)], dtype=np.int32)
_INV = np.array([(q % 16) * 7 + q // 16 for q in range(112)], dtype=np.int32)


def kernel(inputs, hx, sup0_rows, sup0_cols, sup0_vals,
           sup1_rows, sup1_cols, sup1_vals, W_ru, b_ru, W_c, b_c):
  x = jnp.transpose(inputs, (1, 2, 0)).reshape(N, L)

  pad_idx = (jnp.arange(EP - E, dtype=jnp.int32) * 13) % N

  def prep(a, dtype):
    a = a.astype(dtype)
    if dtype == jnp.int32:
      pad = pad_idx
    else:
      pad = jnp.zeros((EP - E,), dtype)
    return jnp.concatenate([a, pad]).reshape(NW, NCHW, CB)

  c0 = prep(sup0_cols, jnp.int32)
  r0 = prep(sup0_rows, jnp.int32)
  v0 = prep(sup0_vals, jnp.float32)
  c1e = prep(sup1_cols, jnp.int32)
  r1e = prep(sup1_rows, jnp.int32)
  v1e = prep(sup1_vals, jnp.float32)

  spmv_x = _make_spmv(1, False)
  spmv_d = _make_spmv(2, True)
  spmv_s = _make_spmv(1, True)
  S1 = spmv_x(x, c0, r0, v0)
  S2, S3, comb1 = spmv_d(S1, c0, r0, v0, c1e, r1e + N, v1e)
  S4, comb3 = spmv_s(S3, c1e, r1e, v1e)
  S5, comb4 = spmv_s(S4, c1e, r1e, v1e)

  bu = b_ru[64:].reshape(1, 64)
  bc = b_c.reshape(1, 64)
  hxp = hx[_INV]

  nblk = pl.BlockSpec((TC_NB, L), lambda i: (i, 0))
  pblk = pl.BlockSpec((NC, TC_NB, L), lambda i: (0, i, 0))
  wublk = pl.BlockSpec((TC_NB, 128), lambda i: (i, 0))
  wcblk = pl.BlockSpec((TC_NB, 64), lambda i: (i, 0))
  cblk = lambda r, c: pl.BlockSpec((r, c), lambda i: (0, 0))

  acc1 = pl.pallas_call(
      _tc1_body,
      grid=(TC_G,),
      in_specs=[nblk, nblk, pblk, wublk, wcblk],
      out_specs=cblk(48, 128),
      out_shape=jax.ShapeDtypeStruct((48, 128), jnp.float32),
      scratch_shapes=[pltpu.VMEM((48, 128), jnp.float32)],
  )(x, comb1, S2, W_ru, W_c)

  new_big = pl.pallas_call(
      _tc2_body,
      grid=(TC_G,),
      in_specs=[
          nblk, nblk, nblk, pblk, wublk, wcblk,
          cblk(48, 128), cblk(1, 64), cblk(1, 64), cblk(112, 64),
      ],
      out_specs=cblk(112, 64),
      out_shape=jax.ShapeDtypeStruct((112, 64), jnp.float32),
      scratch_shapes=[pltpu.VMEM((64, 128), jnp.float32)],
  )(comb1, comb3, comb4, S5, W_ru, W_c, acc1, bu, bc, hxp)
  return new_big[_IDX]
